# Initial kernel scaffold; baseline (speedup 1.0000x reference)
import jax
import jax.numpy as jnp
from jax.experimental import pallas as pl


def kernel(x, edge_index, edge_attr, a_ew0, a_eb0, a_ew1, a_eb1, a_nw0, a_nb0, a_nw1, a_nb1, b_ew0, b_eb0, b_ew1, b_eb1, b_nw0, b_nb0, b_nw1, b_nb1):
    raise NotImplementedError("write your pallas kernel here")


# speedup vs baseline: 2.8750x; 2.8750x over previous
import functools

import numpy as np
import jax
import jax.numpy as jnp
from jax import lax
from jax.experimental import pallas as pl
from jax.experimental.pallas import tpu as pltpu
from jax.experimental.pallas import tpu_sc as plsc

N = 10000
E = 320000
B = 40
NV = 250
NUM_WS = 14
OUT = 128
LR = 0.01

NC = 2
NS = 16
NW = NC * NS
EP = E // NW
C = 400
NPAD = 10240
HALF = NPAD // 2
TROWS = 5248
TSTRIPE = TROWS // NS
OSTRIPE = HALF // NS
EPS = E // NS

BLK = 8000

SQ2 = np.float32(np.sqrt(2.0))


def _lrelu(y):
    return jnp.where(y >= 0, y, 0.2 * y) * SQ2


def _mm_kernel(x_ref, w_ref, o_ref):
    o_ref[...] = jnp.dot(x_ref[...], w_ref[...],
                         preferred_element_type=jnp.float32)


def _matmul(x, w):
    return pl.pallas_call(
        _mm_kernel,
        out_shape=jax.ShapeDtypeStruct((x.shape[0], w.shape[1]), jnp.float32),
    )(x, w)


def _edge1_body(g_ref, ea_ref, we_ref, b0_ref, w2_ref, b1_ref, o_ref):
    ea = ea_ref[...]
    acc = g_ref[...] + b0_ref[...]
    for k in range(4):
        acc = acc + ea[:, k:k + 1] * we_ref[k:k + 1, :]
    h = _lrelu(acc)
    o_ref[...] = _lrelu(jnp.dot(h, w2_ref[...],
                                preferred_element_type=jnp.float32)
                        + b1_ref[...])


def _edge1(g, ea, we, b0, w2, b1):
    grid = (E // BLK,)
    return pl.pallas_call(
        _edge1_body,
        grid=grid,
        in_specs=[
            pl.BlockSpec((BLK, OUT), lambda i: (i, 0)),
            pl.BlockSpec((BLK, 4), lambda i: (i, 0)),
            pl.BlockSpec((4, OUT), lambda i: (0, 0)),
            pl.BlockSpec((1, OUT), lambda i: (0, 0)),
            pl.BlockSpec((OUT, OUT), lambda i: (0, 0)),
            pl.BlockSpec((1, OUT), lambda i: (0, 0)),
        ],
        out_specs=pl.BlockSpec((BLK, OUT), lambda i: (i, 0)),
        out_shape=jax.ShapeDtypeStruct((E, OUT), jnp.float32),
    )(g, ea, we, b0, w2, b1)


def _edge2_body(g_ref, e1_ref, we_ref, b0_ref, w2_ref, b1_ref, o_ref):
    h = _lrelu(g_ref[...] + b0_ref[...]
               + jnp.dot(e1_ref[...], we_ref[...],
                         preferred_element_type=jnp.float32))
    o_ref[...] = _lrelu(jnp.dot(h, w2_ref[...],
                                preferred_element_type=jnp.float32)
                        + b1_ref[...])


def _edge2(g, e1, we, b0, w2, b1):
    grid = (E // BLK,)
    return pl.pallas_call(
        _edge2_body,
        grid=grid,
        in_specs=[
            pl.BlockSpec((BLK, OUT), lambda i: (i, 0)),
            pl.BlockSpec((BLK, OUT), lambda i: (i, 0)),
            pl.BlockSpec((OUT, OUT), lambda i: (0, 0)),
            pl.BlockSpec((1, OUT), lambda i: (0, 0)),
            pl.BlockSpec((OUT, OUT), lambda i: (0, 0)),
            pl.BlockSpec((1, OUT), lambda i: (0, 0)),
        ],
        out_specs=pl.BlockSpec((BLK, OUT), lambda i: (i, 0)),
        out_shape=jax.ShapeDtypeStruct((E, OUT), jnp.float32),
    )(g, e1, we, b0, w2, b1)


def _node1_body(sp_ref, cp_ref, x_ref, a_ref, b_ref, b0_ref, w2_ref, b1_ref,
                x1_ref, cnt_ref):
    sums = sp_ref[...]
    cnt = jnp.maximum(cp_ref[:, 0:1], 1.0)
    agg = sums / cnt
    f = _lrelu(jnp.dot(x_ref[...], a_ref[...],
                       preferred_element_type=jnp.float32)
               + jnp.dot(agg, b_ref[...], preferred_element_type=jnp.float32)
               + b0_ref[...])
    x1_ref[...] = _lrelu(jnp.dot(f, w2_ref[...],
                                 preferred_element_type=jnp.float32)
                         + b1_ref[...])
    cnt_ref[...] = cnt


def _node1(sp, cp, x, a, b, b0, w2, b1):
    return pl.pallas_call(
        _node1_body,
        grid=(1,),
        in_specs=[
            pl.BlockSpec((N, OUT), lambda i: (0, 0)),
            pl.BlockSpec((N, OUT), lambda i: (0, 0)),
            pl.BlockSpec((N, OUT), lambda i: (0, 0)),
            pl.BlockSpec((OUT, OUT), lambda i: (0, 0)),
            pl.BlockSpec((OUT, OUT), lambda i: (0, 0)),
            pl.BlockSpec((1, OUT), lambda i: (0, 0)),
            pl.BlockSpec((OUT, OUT), lambda i: (0, 0)),
            pl.BlockSpec((1, OUT), lambda i: (0, 0)),
        ],
        out_specs=[
            pl.BlockSpec((N, OUT), lambda i: (0, 0)),
            pl.BlockSpec((N, 1), lambda i: (0, 0)),
        ],
        out_shape=[
            jax.ShapeDtypeStruct((N, OUT), jnp.float32),
            jax.ShapeDtypeStruct((N, 1), jnp.float32),
        ],
    )(sp, cp, x, a, b, b0, w2, b1)


def _final_body(x_ref, sp_ref, cnt_ref, a_ref, b_ref, b0_ref, w2_ref, b1_ref,
                o_ref):
    agg = sp_ref[...] / cnt_ref[...]
    f = _lrelu(jnp.dot(x_ref[...], a_ref[...],
                       preferred_element_type=jnp.float32)
               + jnp.dot(agg, b_ref[...], preferred_element_type=jnp.float32)
               + b0_ref[...])
    o_ref[...] = _lrelu(jnp.dot(f, w2_ref[...],
                                preferred_element_type=jnp.float32)
                        + b1_ref[...])


def _final(xg, spg, cg, a, b, b0, w2, b1):
    return pl.pallas_call(
        _final_body,
        out_shape=jax.ShapeDtypeStruct((B, OUT), jnp.float32),
    )(xg, spg, cg, a, b, b0, w2, b1)


def _sc_gather_sum(p, q, s, t):
    mesh = plsc.VectorSubcoreMesh(core_axis_name="c", subcore_axis_name="s")

    @functools.partial(
        pl.kernel, mesh=mesh,
        out_type=jax.ShapeDtypeStruct((E, OUT), jnp.float32),
        scratch_types=[
            pltpu.VMEM((C,), jnp.int32),
            pltpu.VMEM((C,), jnp.int32),
            pltpu.VMEM((C, OUT), jnp.float32),
            pltpu.VMEM((C, OUT), jnp.float32),
            pltpu.SemaphoreType.DMA,
            pltpu.SemaphoreType.DMA,
        ],
    )
    def k(p_hbm, q_hbm, s_hbm, t_hbm, g_hbm, sidx, tidx, bp, bq, sema, semb):
        wid = lax.axis_index("s") * NC + lax.axis_index("c")
        base = wid * EP

        def chunk(i, carry):
            off = base + i * C
            pltpu.sync_copy(s_hbm.at[pl.ds(off, C)], sidx)
            pltpu.sync_copy(t_hbm.at[pl.ds(off, C)], tidx)
            cp = pltpu.async_copy(p_hbm.at[sidx], bp, sema)
            cq = pltpu.async_copy(q_hbm.at[tidx], bq, semb)
            cp.wait()
            cq.wait()

            def add_row(r, c2):
                for g16 in range(OUT // 16):
                    sl = pl.ds(g16 * 16, 16)
                    bp[r, sl] = bp[r, sl] + bq[r, sl]
                return c2

            lax.fori_loop(0, C, add_row, 0, unroll=2)
            pltpu.sync_copy(bp, g_hbm.at[pl.ds(off, C)])
            return carry

        lax.fori_loop(0, EP // C, chunk, 0)

    return k(p, q, s, t)


def _remap(tidx, tidx2, cid):
    base = cid * HALF

    def grp(g16, c2):
        sl = pl.ds(g16 * 16, 16)
        tv = tidx[sl] - base
        ok = (tv >= 0) & (tv < HALF)
        tidx2[sl] = jnp.where(ok, tv, HALF)
        return c2

    lax.fori_loop(0, C // 16, grp, 0)


def _sc_scatter(e, t):
    mesh = plsc.VectorSubcoreMesh(core_axis_name="c", subcore_axis_name="s")

    @functools.partial(
        pl.kernel, mesh=mesh,
        out_type=jax.ShapeDtypeStruct((NPAD, OUT), jnp.float32),
        scratch_types=[
            pltpu.VMEM((C,), jnp.int32),
            pltpu.VMEM((C,), jnp.int32),
            pltpu.VMEM((C, OUT), jnp.float32),
            pltpu.VMEM_SHARED((TROWS, OUT), jnp.float32),
        ],
    )
    def k(e_hbm, t_hbm, out_hbm, tidx, tidx2, buf, table):
        cid = lax.axis_index("c")
        sid = lax.axis_index("s")
        zero16 = jnp.zeros((16,), jnp.float32)

        def zrow(r, c2):
            for g16 in range(OUT // 16):
                buf[r, pl.ds(g16 * 16, 16)] = zero16
            return c2

        lax.fori_loop(0, C, zrow, 0)
        pltpu.sync_copy(buf.at[pl.ds(0, TSTRIPE)],
                        table.at[pl.ds(sid * TSTRIPE, TSTRIPE)])
        plsc.subcore_barrier()

        def chunk(i, c2):
            off = sid * EPS + i * C
            pltpu.sync_copy(t_hbm.at[pl.ds(off, C)], tidx)
            pltpu.sync_copy(e_hbm.at[pl.ds(off, C)], buf)
            _remap(tidx, tidx2, cid)
            pltpu.sync_copy(buf, table.at[tidx2], add=True)
            return c2

        lax.fori_loop(0, EPS // C, chunk, 0)
        plsc.subcore_barrier()
        pltpu.sync_copy(table.at[pl.ds(sid * OSTRIPE, OSTRIPE)],
                        out_hbm.at[pl.ds(cid * HALF + sid * OSTRIPE, OSTRIPE)])

    return k(e, t)


def _sc_count(t):
    mesh = plsc.VectorSubcoreMesh(core_axis_name="c", subcore_axis_name="s")

    @functools.partial(
        pl.kernel, mesh=mesh,
        out_type=jax.ShapeDtypeStruct((NPAD, OUT), jnp.float32),
        scratch_types=[
            pltpu.VMEM((C,), jnp.int32),
            pltpu.VMEM((C,), jnp.int32),
            pltpu.VMEM((C, OUT), jnp.float32),
            pltpu.VMEM_SHARED((TROWS, OUT), jnp.float32),
        ],
    )
    def k(t_hbm, out_hbm, tidx, tidx2, buf, table):
        cid = lax.axis_index("c")
        sid = lax.axis_index("s")
        zero16 = jnp.zeros((16,), jnp.float32)

        def zrow(r, c2):
            for g16 in range(OUT // 16):
                buf[r, pl.ds(g16 * 16, 16)] = zero16
            return c2

        lax.fori_loop(0, C, zrow, 0)
        pltpu.sync_copy(buf.at[pl.ds(0, TSTRIPE)],
                        table.at[pl.ds(sid * TSTRIPE, TSTRIPE)])
        lane = lax.iota(jnp.int32, 16)
        one16 = jnp.where(lane == 0, 1.0, 0.0).astype(jnp.float32)

        def orow(r, c2):
            buf[r, pl.ds(0, 16)] = one16
            return c2

        lax.fori_loop(0, C, orow, 0)
        plsc.subcore_barrier()

        def chunk(i, c2):
            off = sid * EPS + i * C
            pltpu.sync_copy(t_hbm.at[pl.ds(off, C)], tidx)
            _remap(tidx, tidx2, cid)
            pltpu.sync_copy(buf, table.at[tidx2], add=True)
            return c2

        lax.fori_loop(0, EPS // C, chunk, 0)
        plsc.subcore_barrier()
        pltpu.sync_copy(table.at[pl.ds(sid * OSTRIPE, OSTRIPE)],
                        out_hbm.at[pl.ds(cid * HALF + sid * OSTRIPE, OSTRIPE)])

    return k(t)


def kernel(x, edge_index, edge_attr,
           a_ew0, a_eb0, a_ew1, a_eb1, a_nw0, a_nb0, a_nw1, a_nb1,
           b_ew0, b_eb0, b_ew1, b_eb1, b_nw0, b_nb0, b_nw1, b_nb1):
    s = edge_index[0]
    t = edge_index[1]

    def prep_edge(ew0, eb0, ew1, eb1, kin):
        w = (ew0 * np.float32(LR / np.sqrt(kin))).T
        pq_w = jnp.concatenate([w[:OUT], w[OUT:2 * OUT]], axis=1)
        we = w[2 * OUT:]
        w2 = (ew1 * np.float32(LR / np.sqrt(OUT))).T
        return pq_w, we, (eb0 * LR)[None, :], w2, (eb1 * LR)[None, :]

    def prep_node(nw0, nb0, nw1, nb1):
        w = (nw0 * np.float32(LR / np.sqrt(2 * OUT))).T
        w2 = (nw1 * np.float32(LR / np.sqrt(OUT))).T
        return w[:OUT], w[OUT:], (nb0 * LR)[None, :], w2, (nb1 * LR)[None, :]

    pq_w1, we1, e1b0, w1e2, e1b1 = prep_edge(a_ew0, a_eb0, a_ew1, a_eb1, 260)
    n1a, n1b, n1b0, n1w2, n1b1 = prep_node(a_nw0, a_nb0, a_nw1, a_nb1)
    pq_w2, we2, e2b0, w2e2, e2b1 = prep_edge(b_ew0, b_eb0, b_ew1, b_eb1, 384)
    n2a, n2b, n2b0, n2w2, n2b1 = prep_node(b_nw0, b_nb0, b_nw1, b_nb1)

    cp = _sc_count(t)
    pq1 = _matmul(x, pq_w1)
    g1 = _sc_gather_sum(pq1[:, :OUT], pq1[:, OUT:], s, t)
    e1 = _edge1(g1, edge_attr, we1, e1b0, w1e2, e1b1)
    sp1 = _sc_scatter(e1, t)
    x1, cnt = _node1(sp1, cp, x, n1a, n1b, n1b0, n1w2, n1b1)

    pq2 = _matmul(x1, pq_w2)
    g2 = _sc_gather_sum(pq2[:, :OUT], pq2[:, OUT:], s, t)
    e2 = _edge2(g2, e1, we2, e2b0, w2e2, e2b1)
    sp2 = _sc_scatter(e2, t)

    xg = lax.slice(x1, (0, 0), ((B - 1) * NV + 1, OUT), (NV, 1))
    spg = lax.slice(sp2, (0, 0), ((B - 1) * NV + 1, OUT), (NV, 1))
    cg = lax.slice(cnt, (0, 0), ((B - 1) * NV + 1, 1), (NV, 1))
    x2g = _final(xg, spg, cg, n2a, n2b, n2b0, n2w2, n2b1)
    return jnp.broadcast_to(x2g[:, None, :], (B, NUM_WS, OUT))
